# baseline (speedup 1.0000x reference)
import functools

import jax
import jax.numpy as jnp
from jax import lax
from jax.experimental import pallas as pl
from jax.experimental.pallas import tpu as pltpu
from jax.experimental.pallas import tpu_sc as plsc

N = 10000
G = 128
E = 320000

N_PAD = 10112
D = 128
NC, NS, LANES = 2, 16, 16
NW = NC * NS
CHUNK = 128
CPW = 84
EPW = CPW * CHUNK
E_PAD = NW * EPW
ROWS_PER_SUB = N_PAD // NS
RBLK = 1264

_MESH = plsc.VectorSubcoreMesh(core_axis_name="c", subcore_axis_name="s")


def _sc_hist(src_hist, consts):
    @functools.partial(
        pl.kernel,
        out_type=jax.ShapeDtypeStruct((NC, N_PAD, D), jnp.float32),
        mesh=_MESH,
        scratch_types=[
            pltpu.VMEM((CHUNK,), jnp.int32),
            pltpu.VMEM((CHUNK,), jnp.int32),
            pltpu.VMEM((CHUNK, D), jnp.float32),
            pltpu.VMEM_SHARED((N_PAD, D), jnp.float32),
            pltpu.SemaphoreType.DMA,
            pltpu.SemaphoreType.DMA,
        ],
    )
    def k(idx_hbm, const_hbm, out_hbm, idx_c, idx_d, buf, acc, sem0, sem1):
        c = lax.axis_index("c")
        s = lax.axis_index("s")
        w = s * NC + c
        pltpu.sync_copy(const_hbm.at[0], buf)
        base = s * ROWS_PER_SUB

        @pl.loop(0, ROWS_PER_SUB // CHUNK)
        def _(t):
            pltpu.sync_copy(buf, acc.at[pl.ds(base + t * CHUNK, CHUNK)])

        pltpu.sync_copy(
            buf.at[pl.ds(0, ROWS_PER_SUB % CHUNK)],
            acc.at[pl.ds(base + (ROWS_PER_SUB // CHUNK) * CHUNK,
                         ROWS_PER_SUB % CHUNK)])
        pltpu.sync_copy(const_hbm.at[1], buf)
        plsc.subcore_barrier()

        @pl.loop(0, CPW // 2)
        def _(p):
            j = p * 2
            pltpu.sync_copy(idx_hbm.at[pl.ds(w * EPW + j * CHUNK, CHUNK)], idx_c)
            pltpu.sync_copy(idx_hbm.at[pl.ds(w * EPW + (j + 1) * CHUNK, CHUNK)],
                            idx_d)
            s0 = pltpu.async_copy(buf, acc.at[idx_c], sem0, add=True)
            s1 = pltpu.async_copy(buf, acc.at[idx_d], sem1, add=True)
            s0.wait()
            s1.wait()

        plsc.subcore_barrier()
        pltpu.sync_copy(acc.at[pl.ds(base, ROWS_PER_SUB)],
                        out_hbm.at[c, pl.ds(base, ROWS_PER_SUB)])

    return k(src_hist, consts)


def _sc_prop(table, src_flat, dst_flat, consts):
    @functools.partial(
        pl.kernel,
        out_type=jax.ShapeDtypeStruct((NC, N_PAD, D), jnp.float32),
        mesh=_MESH,
        scratch_types=[
            pltpu.VMEM((CHUNK,), jnp.int32),
            pltpu.VMEM((CHUNK,), jnp.int32),
            pltpu.VMEM((CHUNK,), jnp.int32),
            pltpu.VMEM((CHUNK,), jnp.int32),
            pltpu.VMEM((CHUNK,), jnp.int32),
            pltpu.VMEM((CHUNK,), jnp.int32),
            pltpu.VMEM((CHUNK, D), jnp.float32),
            pltpu.VMEM((CHUNK, D), jnp.float32),
            pltpu.VMEM((CHUNK, D), jnp.float32),
            pltpu.VMEM_SHARED((N_PAD, D), jnp.float32),
            pltpu.SemaphoreType.DMA,
            pltpu.SemaphoreType.DMA,
            pltpu.SemaphoreType.DMA,
            pltpu.SemaphoreType.DMA,
            pltpu.SemaphoreType.DMA,
            pltpu.SemaphoreType.DMA,
        ],
    )
    def k(table_hbm, src_hbm, dst_hbm, const_hbm, out_hbm,
          sidx0, sidx1, sidx2, didx0, didx1, didx2, buf0, buf1, buf2, acc,
          sem0, sem1, sem2, sem3, sem4, sem5):
        c = lax.axis_index("c")
        s = lax.axis_index("s")
        w = s * NC + c
        pltpu.sync_copy(const_hbm.at[0], buf0)
        base = s * ROWS_PER_SUB

        @pl.loop(0, ROWS_PER_SUB // CHUNK)
        def _(t):
            pltpu.sync_copy(buf0, acc.at[pl.ds(base + t * CHUNK, CHUNK)])

        pltpu.sync_copy(
            buf0.at[pl.ds(0, ROWS_PER_SUB % CHUNK)],
            acc.at[pl.ds(base + (ROWS_PER_SUB // CHUNK) * CHUNK,
                         ROWS_PER_SUB % CHUNK)])
        plsc.subcore_barrier()
        ebase = w * EPW

        @pl.loop(0, CPW // 3)
        def _(p):
            j = p * 3
            pltpu.sync_copy(src_hbm.at[pl.ds(ebase + j * CHUNK, CHUNK)], sidx0)
            g0 = pltpu.async_copy(table_hbm.at[sidx0], buf0, sem0)
            pltpu.sync_copy(src_hbm.at[pl.ds(ebase + (j + 1) * CHUNK, CHUNK)],
                            sidx1)
            g1 = pltpu.async_copy(table_hbm.at[sidx1], buf1, sem1)
            pltpu.sync_copy(src_hbm.at[pl.ds(ebase + (j + 2) * CHUNK, CHUNK)],
                            sidx2)
            g2 = pltpu.async_copy(table_hbm.at[sidx2], buf2, sem2)
            pltpu.sync_copy(dst_hbm.at[pl.ds(ebase + j * CHUNK, CHUNK)], didx0)
            pltpu.sync_copy(dst_hbm.at[pl.ds(ebase + (j + 1) * CHUNK, CHUNK)],
                            didx1)
            pltpu.sync_copy(dst_hbm.at[pl.ds(ebase + (j + 2) * CHUNK, CHUNK)],
                            didx2)
            g0.wait()
            s0 = pltpu.async_copy(buf0, acc.at[didx0], sem3, add=True)
            g1.wait()
            s1 = pltpu.async_copy(buf1, acc.at[didx1], sem4, add=True)
            g2.wait()
            s2 = pltpu.async_copy(buf2, acc.at[didx2], sem5, add=True)
            s0.wait()
            s1.wait()
            s2.wait()

        plsc.subcore_barrier()
        pltpu.sync_copy(acc.at[pl.ds(base, ROWS_PER_SUB)],
                        out_hbm.at[c, pl.ds(base, ROWS_PER_SUB)])

    return k(table, src_flat, dst_flat, consts)


def _tc_matmul(xp, wcat):
    kdim, m = wcat.shape

    def body(x_ref, w_ref, o_ref):
        o_ref[...] = jnp.dot(x_ref[...], w_ref[...],
                             preferred_element_type=jnp.float32)

    return pl.pallas_call(
        body,
        grid=(N_PAD // RBLK,),
        in_specs=[pl.BlockSpec((RBLK, kdim), lambda i: (i, 0)),
                  pl.BlockSpec((kdim, m), lambda i: (0, 0))],
        out_specs=pl.BlockSpec((RBLK, m), lambda i: (i, 0)),
        out_shape=jax.ShapeDtypeStruct((N_PAD, m), jnp.float32),
    )(xp, wcat)


def _tc_dis_g2(degp, a):
    def body(d_ref, a_ref, dis_ref, g2_ref):
        deg = d_ref[0, :, 0:1] + d_ref[1, :, 0:1]
        dis = jnp.where(deg > 0, lax.rsqrt(jnp.maximum(deg, 1e-12)), 0.0)
        dis_ref[...] = dis
        g2_ref[...] = dis * a_ref[...]

    return pl.pallas_call(
        body,
        grid=(N_PAD // RBLK,),
        in_specs=[pl.BlockSpec((NC, RBLK, D), lambda i: (0, i, 0)),
                  pl.BlockSpec((RBLK, 128), lambda i: (i, 2))],
        out_specs=[pl.BlockSpec((RBLK, 1), lambda i: (i, 0)),
                   pl.BlockSpec((RBLK, 128), lambda i: (i, 0))],
        out_shape=[jax.ShapeDtypeStruct((N_PAD, 1), jnp.float32),
                   jax.ShapeDtypeStruct((N_PAD, 128), jnp.float32)],
    )(degp, a)


def _tc_g1(dis, a, s2):
    def body(dis_ref, a_ref, s_ref, o_ref):
        dis_b = dis_ref[...]
        o_ref[...] = dis_b * (a_ref[...] - dis_b * (s_ref[0] + s_ref[1]))

    return pl.pallas_call(
        body,
        grid=(N_PAD // RBLK,),
        in_specs=[pl.BlockSpec((RBLK, 1), lambda i: (i, 0)),
                  pl.BlockSpec((RBLK, 128), lambda i: (i, 1)),
                  pl.BlockSpec((NC, RBLK, 128), lambda i: (0, i, 0))],
        out_specs=pl.BlockSpec((RBLK, 128), lambda i: (i, 0)),
        out_shape=jax.ShapeDtypeStruct((N_PAD, 128), jnp.float32),
    )(dis, a, s2)


def _tc_h_mm(dis, a, s1, b1t, vcat):
    m = vcat.shape[1]

    def body(dis_ref, a_ref, s_ref, b_ref, w_ref, c_ref, g_ref):
        dis_b = dis_ref[...]
        h = a_ref[...] + b_ref[0:1, :] - dis_b * (s_ref[0] + s_ref[1])
        h = jnp.maximum(h, 0.0)
        cfull = jnp.dot(h, w_ref[...], preferred_element_type=jnp.float32)
        c_ref[...] = cfull
        g_ref[...] = jnp.concatenate(
            [dis_b * cfull[:, 128:], jnp.zeros((RBLK, 64), jnp.float32)],
            axis=1)

    return pl.pallas_call(
        body,
        grid=(N_PAD // RBLK,),
        in_specs=[pl.BlockSpec((RBLK, 1), lambda i: (i, 0)),
                  pl.BlockSpec((RBLK, 128), lambda i: (i, 0)),
                  pl.BlockSpec((NC, RBLK, 128), lambda i: (0, i, 0)),
                  pl.BlockSpec((8, 128), lambda i: (0, 0)),
                  pl.BlockSpec((128, m), lambda i: (0, 0))],
        out_specs=[pl.BlockSpec((RBLK, m), lambda i: (i, 0)),
                   pl.BlockSpec((RBLK, 128), lambda i: (i, 0))],
        out_shape=[jax.ShapeDtypeStruct((N_PAD, m), jnp.float32),
                   jax.ShapeDtypeStruct((N_PAD, 128), jnp.float32)],
    )(dis, a, s1, b1t, vcat)


def _tc_g1p(dis, cmat, s2p):
    def body(dis_ref, c_ref, s_ref, o_ref):
        dis_b = dis_ref[...]
        val = dis_b * (c_ref[:, 64:128]
                       - dis_b * (s_ref[0, :, 0:64] + s_ref[1, :, 0:64]))
        o_ref[...] = jnp.concatenate(
            [val, jnp.zeros((RBLK, 64), jnp.float32)], axis=1)

    return pl.pallas_call(
        body,
        grid=(N_PAD // RBLK,),
        in_specs=[pl.BlockSpec((RBLK, 1), lambda i: (i, 0)),
                  pl.BlockSpec((RBLK, 192), lambda i: (i, 0)),
                  pl.BlockSpec((NC, RBLK, 128), lambda i: (0, i, 0))],
        out_specs=pl.BlockSpec((RBLK, 128), lambda i: (i, 0)),
        out_shape=jax.ShapeDtypeStruct((N_PAD, 128), jnp.float32),
    )(dis, cmat, s2p)


def _tc_pool(dis, cmat, s1p, b2t, batch2d):
    nblk = N_PAD // RBLK

    def body(dis_ref, c_ref, s_ref, b_ref, bt_ref, o_ref, pool_acc, cnt_acc):
        i = pl.program_id(0)
        dis_b = dis_ref[...]
        o_blk = (c_ref[:, 0:64] + b_ref[0:1, :]
                 - dis_b * (s_ref[0, :, 0:64] + s_ref[1, :, 0:64]))
        ids = bt_ref[...]
        iota = lax.broadcasted_iota(jnp.int32, (RBLK, G), 1)
        onehot = (iota == ids).astype(jnp.float32)
        pp = lax.dot_general(onehot, o_blk, (((0,), (0,)), ((), ())),
                             preferred_element_type=jnp.float32)
        cc = jnp.sum(onehot, axis=0)[None, :]

        @pl.when(i == 0)
        def _():
            pool_acc[...] = pp
            cnt_acc[...] = cc

        @pl.when(i > 0)
        def _():
            pool_acc[...] += pp
            cnt_acc[...] += cc

        @pl.when(i == nblk - 1)
        def _():
            cnt = jnp.maximum(cnt_acc[...], 1.0)
            p = pool_acc[...] / cnt.reshape(G, 1)
            m = jnp.max(p, axis=1, keepdims=True)
            z = p - m
            o_ref[...] = z - jnp.log(jnp.sum(jnp.exp(z), axis=1, keepdims=True))

    return pl.pallas_call(
        body,
        grid=(nblk,),
        in_specs=[pl.BlockSpec((RBLK, 1), lambda i: (i, 0)),
                  pl.BlockSpec((RBLK, 192), lambda i: (i, 0)),
                  pl.BlockSpec((NC, RBLK, 128), lambda i: (0, i, 0)),
                  pl.BlockSpec((8, 64), lambda i: (0, 0)),
                  pl.BlockSpec((RBLK, 1), lambda i: (i, 0))],
        out_specs=pl.BlockSpec((G, 64), lambda i: (0, 0)),
        out_shape=jax.ShapeDtypeStruct((G, 64), jnp.float32),
        scratch_shapes=[pltpu.VMEM((G, 64), jnp.float32),
                        pltpu.VMEM((1, G), jnp.float32)],
    )(dis, cmat, s1p, b2t, batch2d)


def kernel(x, edge_index, batch, W1, b1, W2, b2):
    src = edge_index[0].astype(jnp.int32)
    dst = edge_index[1].astype(jnp.int32)
    pad_e = E_PAD - E
    pad_cycle = jnp.arange(pad_e, dtype=jnp.int32)
    pad_gather = pad_cycle % N
    pad_trash = N + (pad_cycle % (N_PAD - N))
    src_gather = jnp.concatenate([src, pad_gather])
    dst_scatter = jnp.concatenate([dst, pad_trash])
    src_hist = jnp.concatenate([src, pad_trash])

    xp = jnp.pad(x, ((0, N_PAD - N), (0, 0)))
    batch2d = jnp.concatenate(
        [batch.astype(jnp.int32), jnp.full((N_PAD - N,), G, jnp.int32)])[:, None]

    ucat = jnp.concatenate([W1[0] - W1[2], W1[1], 2.0 * W1[2]], axis=1)
    vcat = jnp.concatenate([W2[0] - W2[2], W2[1], 2.0 * W2[2]], axis=1)
    b1t = jnp.broadcast_to(b1, (8, 128))
    b2t = jnp.broadcast_to(b2, (8, 64))
    consts = jnp.stack([jnp.zeros((CHUNK, D), jnp.float32),
                        jnp.ones((CHUNK, D), jnp.float32)])

    degp = _sc_hist(src_hist, consts)
    a = _tc_matmul(xp, ucat)
    dis, g2 = _tc_dis_g2(degp, a)
    s2 = _sc_prop(g2, src_gather, dst_scatter, consts)
    g1 = _tc_g1(dis, a, s2)
    s1 = _sc_prop(g1, src_gather, dst_scatter, consts)
    cmat, g2p = _tc_h_mm(dis, a, s1, b1t, vcat)
    s2p = _sc_prop(g2p, src_gather, dst_scatter, consts)
    g1p = _tc_g1p(dis, cmat, s2p)
    s1p = _sc_prop(g1p, src_gather, dst_scatter, consts)
    return _tc_pool(dis, cmat, s1p, b2t, batch2d)


# speedup vs baseline: 1.1047x; 1.1047x over previous
import functools

import jax
import jax.numpy as jnp
from jax import lax
from jax.experimental import pallas as pl
from jax.experimental.pallas import tpu as pltpu
from jax.experimental.pallas import tpu_sc as plsc

N = 10000
G = 128
E = 320000

N_PAD = 10240
TRASH = 10200
D = 128
NC, NS, LANES = 2, 16, 16
NW = NC * NS
CHUNK = 128
CPW = 80
EPW = CPW * CHUNK
E_PAD = NW * EPW
ROWS_PER_SUB = N_PAD // NS
RBLK = 2048

_MESH = plsc.VectorSubcoreMesh(core_axis_name="c", subcore_axis_name="s")


def _sc_hist(src_hist, consts):
    @functools.partial(
        pl.kernel,
        out_type=jax.ShapeDtypeStruct((NC, N_PAD, D), jnp.float32),
        mesh=_MESH,
        scratch_types=[
            pltpu.VMEM((CHUNK,), jnp.int32),
            pltpu.VMEM((CHUNK,), jnp.int32),
            pltpu.VMEM((CHUNK, D), jnp.float32),
            pltpu.VMEM_SHARED((N_PAD, D), jnp.float32),
            pltpu.SemaphoreType.DMA,
            pltpu.SemaphoreType.DMA,
        ],
    )
    def k(idx_hbm, const_hbm, out_hbm, idx_c, idx_d, buf, acc, sem0, sem1):
        c = lax.axis_index("c")
        s = lax.axis_index("s")
        w = s * NC + c
        pltpu.sync_copy(const_hbm.at[0], buf)
        base = s * ROWS_PER_SUB

        @pl.loop(0, ROWS_PER_SUB // CHUNK)
        def _(t):
            pltpu.sync_copy(buf, acc.at[pl.ds(base + t * CHUNK, CHUNK)])

        pltpu.sync_copy(const_hbm.at[1], buf)
        plsc.subcore_barrier()

        @pl.loop(0, CPW // 2)
        def _(p):
            j = p * 2
            pltpu.sync_copy(idx_hbm.at[pl.ds(w * EPW + j * CHUNK, CHUNK)], idx_c)
            pltpu.sync_copy(idx_hbm.at[pl.ds(w * EPW + (j + 1) * CHUNK, CHUNK)],
                            idx_d)
            s0 = pltpu.async_copy(buf, acc.at[idx_c], sem0, add=True)
            s1 = pltpu.async_copy(buf, acc.at[idx_d], sem1, add=True)
            s0.wait()
            s1.wait()

        plsc.subcore_barrier()
        pltpu.sync_copy(acc.at[pl.ds(base, ROWS_PER_SUB)],
                        out_hbm.at[c, pl.ds(base, ROWS_PER_SUB)])

    return k(src_hist, consts)


def _sc_prop(table, src_flat, dst_flat, consts):
    @functools.partial(
        pl.kernel,
        out_type=jax.ShapeDtypeStruct((NC, N_PAD, D), jnp.float32),
        mesh=_MESH,
        scratch_types=[
            pltpu.VMEM((CPW, CHUNK), jnp.int32),
            pltpu.VMEM((CHUNK,), jnp.int32),
            pltpu.VMEM((CHUNK,), jnp.int32),
            pltpu.VMEM((CHUNK, D), jnp.float32),
            pltpu.VMEM((CHUNK, D), jnp.float32),
            pltpu.VMEM_SHARED((N_PAD, D), jnp.float32),
            pltpu.SemaphoreType.DMA,
            pltpu.SemaphoreType.DMA,
            pltpu.SemaphoreType.DMA,
            pltpu.SemaphoreType.DMA,
        ],
    )
    def k(table_hbm, src_hbm, dst_hbm, const_hbm, out_hbm,
          sidx_all, didx0, didx1, buf0, buf1, acc, sem0, sem1, sem2, sem3):
        c = lax.axis_index("c")
        s = lax.axis_index("s")
        w = s * NC + c
        pltpu.sync_copy(const_hbm.at[0], buf0)
        base = s * ROWS_PER_SUB

        @pl.loop(0, ROWS_PER_SUB // CHUNK)
        def _(t):
            pltpu.sync_copy(buf0, acc.at[pl.ds(base + t * CHUNK, CHUNK)])

        pltpu.sync_copy(src_hbm.at[w], sidx_all)
        plsc.subcore_barrier()
        ebase = w * EPW

        @pl.loop(0, CPW // 2)
        def _(p):
            j = p * 2
            g0 = pltpu.async_copy(table_hbm.at[sidx_all.at[j]], buf0, sem0)
            g1 = pltpu.async_copy(table_hbm.at[sidx_all.at[j + 1]], buf1, sem1)
            pltpu.sync_copy(dst_hbm.at[pl.ds(ebase + j * CHUNK, CHUNK)], didx0)
            pltpu.sync_copy(dst_hbm.at[pl.ds(ebase + (j + 1) * CHUNK, CHUNK)],
                            didx1)
            g0.wait()
            s0 = pltpu.async_copy(buf0, acc.at[didx0], sem2, add=True)
            g1.wait()
            s1 = pltpu.async_copy(buf1, acc.at[didx1], sem3, add=True)
            s0.wait()
            s1.wait()

        plsc.subcore_barrier()
        pltpu.sync_copy(acc.at[pl.ds(base, ROWS_PER_SUB)],
                        out_hbm.at[c, pl.ds(base, ROWS_PER_SUB)])

    return k(table, src_flat.reshape(NW, CPW, CHUNK), dst_flat, consts)


def _tc_matmul(xp, wcat):
    kdim, m = wcat.shape

    def body(x_ref, w_ref, o_ref):
        o_ref[...] = jnp.dot(x_ref[...], w_ref[...],
                             preferred_element_type=jnp.float32)

    return pl.pallas_call(
        body,
        grid=(N_PAD // RBLK,),
        in_specs=[pl.BlockSpec((RBLK, kdim), lambda i: (i, 0)),
                  pl.BlockSpec((kdim, m), lambda i: (0, 0))],
        out_specs=pl.BlockSpec((RBLK, m), lambda i: (i, 0)),
        out_shape=jax.ShapeDtypeStruct((N_PAD, m), jnp.float32),
    )(xp, wcat)


def _tc_dis_g2(degp, a):
    def body(d_ref, a_ref, dis_ref, g2_ref):
        deg = d_ref[0, :, 0:1] + d_ref[1, :, 0:1]
        dis = jnp.where(deg > 0, lax.rsqrt(jnp.maximum(deg, 1e-12)), 0.0)
        dis_ref[...] = dis
        g2_ref[...] = dis * a_ref[...]

    return pl.pallas_call(
        body,
        grid=(N_PAD // RBLK,),
        in_specs=[pl.BlockSpec((NC, RBLK, D), lambda i: (0, i, 0)),
                  pl.BlockSpec((RBLK, 128), lambda i: (i, 2))],
        out_specs=[pl.BlockSpec((RBLK, 1), lambda i: (i, 0)),
                   pl.BlockSpec((RBLK, 128), lambda i: (i, 0))],
        out_shape=[jax.ShapeDtypeStruct((N_PAD, 1), jnp.float32),
                   jax.ShapeDtypeStruct((N_PAD, 128), jnp.float32)],
    )(degp, a)


def _tc_g1(dis, a, s2):
    def body(dis_ref, a_ref, s_ref, o_ref):
        dis_b = dis_ref[...]
        o_ref[...] = dis_b * (a_ref[...] - dis_b * (s_ref[0] + s_ref[1]))

    return pl.pallas_call(
        body,
        grid=(N_PAD // RBLK,),
        in_specs=[pl.BlockSpec((RBLK, 1), lambda i: (i, 0)),
                  pl.BlockSpec((RBLK, 128), lambda i: (i, 1)),
                  pl.BlockSpec((NC, RBLK, 128), lambda i: (0, i, 0))],
        out_specs=pl.BlockSpec((RBLK, 128), lambda i: (i, 0)),
        out_shape=jax.ShapeDtypeStruct((N_PAD, 128), jnp.float32),
    )(dis, a, s2)


def _tc_h_mm(dis, a, s1, b1t, vcat):
    m = vcat.shape[1]

    def body(dis_ref, a_ref, s_ref, b_ref, w_ref, c_ref, g_ref):
        dis_b = dis_ref[...]
        h = a_ref[...] + b_ref[0:1, :] - dis_b * (s_ref[0] + s_ref[1])
        h = jnp.maximum(h, 0.0)
        cfull = jnp.dot(h, w_ref[...], preferred_element_type=jnp.float32)
        c_ref[...] = cfull
        g_ref[...] = jnp.concatenate(
            [dis_b * cfull[:, 128:], jnp.zeros((RBLK, 64), jnp.float32)],
            axis=1)

    return pl.pallas_call(
        body,
        grid=(N_PAD // RBLK,),
        in_specs=[pl.BlockSpec((RBLK, 1), lambda i: (i, 0)),
                  pl.BlockSpec((RBLK, 128), lambda i: (i, 0)),
                  pl.BlockSpec((NC, RBLK, 128), lambda i: (0, i, 0)),
                  pl.BlockSpec((8, 128), lambda i: (0, 0)),
                  pl.BlockSpec((128, m), lambda i: (0, 0))],
        out_specs=[pl.BlockSpec((RBLK, m), lambda i: (i, 0)),
                   pl.BlockSpec((RBLK, 128), lambda i: (i, 0))],
        out_shape=[jax.ShapeDtypeStruct((N_PAD, m), jnp.float32),
                   jax.ShapeDtypeStruct((N_PAD, 128), jnp.float32)],
    )(dis, a, s1, b1t, vcat)


def _tc_g1p(dis, cmat, s2p):
    def body(dis_ref, c_ref, s_ref, o_ref):
        dis_b = dis_ref[...]
        val = dis_b * (c_ref[:, 64:128]
                       - dis_b * (s_ref[0, :, 0:64] + s_ref[1, :, 0:64]))
        o_ref[...] = jnp.concatenate(
            [val, jnp.zeros((RBLK, 64), jnp.float32)], axis=1)

    return pl.pallas_call(
        body,
        grid=(N_PAD // RBLK,),
        in_specs=[pl.BlockSpec((RBLK, 1), lambda i: (i, 0)),
                  pl.BlockSpec((RBLK, 192), lambda i: (i, 0)),
                  pl.BlockSpec((NC, RBLK, 128), lambda i: (0, i, 0))],
        out_specs=pl.BlockSpec((RBLK, 128), lambda i: (i, 0)),
        out_shape=jax.ShapeDtypeStruct((N_PAD, 128), jnp.float32),
    )(dis, cmat, s2p)


def _tc_pool(dis, cmat, s1p, b2t, batch2d):
    nblk = N_PAD // RBLK

    def body(dis_ref, c_ref, s_ref, b_ref, bt_ref, o_ref, pool_acc, cnt_acc):
        i = pl.program_id(0)
        dis_b = dis_ref[...]
        o_blk = (c_ref[:, 0:64] + b_ref[0:1, :]
                 - dis_b * (s_ref[0, :, 0:64] + s_ref[1, :, 0:64]))
        ids = bt_ref[...]
        iota = lax.broadcasted_iota(jnp.int32, (RBLK, G), 1)
        onehot = (iota == ids).astype(jnp.float32)
        pp = lax.dot_general(onehot, o_blk, (((0,), (0,)), ((), ())),
                             preferred_element_type=jnp.float32)
        cc = jnp.sum(onehot, axis=0)[None, :]

        @pl.when(i == 0)
        def _():
            pool_acc[...] = pp
            cnt_acc[...] = cc

        @pl.when(i > 0)
        def _():
            pool_acc[...] += pp
            cnt_acc[...] += cc

        @pl.when(i == nblk - 1)
        def _():
            cnt = jnp.maximum(cnt_acc[...], 1.0)
            p = pool_acc[...] / cnt.reshape(G, 1)
            m = jnp.max(p, axis=1, keepdims=True)
            z = p - m
            o_ref[...] = z - jnp.log(jnp.sum(jnp.exp(z), axis=1, keepdims=True))

    return pl.pallas_call(
        body,
        grid=(nblk,),
        in_specs=[pl.BlockSpec((RBLK, 1), lambda i: (i, 0)),
                  pl.BlockSpec((RBLK, 192), lambda i: (i, 0)),
                  pl.BlockSpec((NC, RBLK, 128), lambda i: (0, i, 0)),
                  pl.BlockSpec((8, 64), lambda i: (0, 0)),
                  pl.BlockSpec((RBLK, 1), lambda i: (i, 0))],
        out_specs=pl.BlockSpec((G, 64), lambda i: (0, 0)),
        out_shape=jax.ShapeDtypeStruct((G, 64), jnp.float32),
        scratch_shapes=[pltpu.VMEM((G, 64), jnp.float32),
                        pltpu.VMEM((1, G), jnp.float32)],
    )(dis, cmat, s1p, b2t, batch2d)


def kernel(x, edge_index, batch, W1, b1, W2, b2):
    src = edge_index[0].astype(jnp.int32)
    dst = edge_index[1].astype(jnp.int32)
    pad_e = E_PAD - E
    pad_cycle = jnp.arange(pad_e, dtype=jnp.int32)
    pad_gather = pad_cycle % N
    pad_trash = N + (pad_cycle % (N_PAD - N))
    src_gather = jnp.concatenate([src, pad_gather])
    dst_scatter = jnp.concatenate([dst, pad_trash])
    src_hist = jnp.concatenate([src, pad_trash])

    xp = jnp.pad(x, ((0, N_PAD - N), (0, 0)))
    batch2d = jnp.concatenate(
        [batch.astype(jnp.int32), jnp.full((N_PAD - N,), G, jnp.int32)])[:, None]

    ucat = jnp.concatenate([W1[0] - W1[2], W1[1], 2.0 * W1[2]], axis=1)
    vcat = jnp.concatenate([W2[0] - W2[2], W2[1], 2.0 * W2[2]], axis=1)
    b1t = jnp.broadcast_to(b1, (8, 128))
    b2t = jnp.broadcast_to(b2, (8, 64))
    consts = jnp.stack([jnp.zeros((CHUNK, D), jnp.float32),
                        jnp.ones((CHUNK, D), jnp.float32)])

    degp = _sc_hist(src_hist, consts)
    a = _tc_matmul(xp, ucat)
    dis, g2 = _tc_dis_g2(degp, a)
    s2 = _sc_prop(g2, src_gather, dst_scatter, consts)
    g1 = _tc_g1(dis, a, s2)
    s1 = _sc_prop(g1, src_gather, dst_scatter, consts)
    cmat, g2p = _tc_h_mm(dis, a, s1, b1t, vcat)
    s2p = _sc_prop(g2p, src_gather, dst_scatter, consts)
    g1p = _tc_g1p(dis, cmat, s2p)
    s1p = _sc_prop(g1p, src_gather, dst_scatter, consts)
    return _tc_pool(dis, cmat, s1p, b2t, batch2d)
